# baseline (speedup 1.0000x reference)
import numpy as np
import jax
import jax.numpy as jnp
from jax import lax
from jax.experimental import pallas as pl
from jax.experimental.pallas import tpu as pltpu
from jax.experimental.pallas import tpu_sc as plsc

N = 10000
E = 320000
IN_DIM = 128
H = 8
D = 16
HD = H * D
S12 = 16

NC = 2
NS = 16
NW = NC * NS
EPW = E // NW
CHUNK = 40
NCHUNK = EPW // CHUNK
RPT = 624


def _mm_body(x_ref, w_ref, a12_ref, h_ref, s12_ref):
    hb = jnp.dot(x_ref[...], w_ref[...], preferred_element_type=jnp.float32)
    h_ref[...] = hb
    s12_ref[...] = jnp.dot(hb, a12_ref[...], preferred_element_type=jnp.float32)


_MM_B = 1000


def _stage1(x, W, A12):
    return pl.pallas_call(
        _mm_body,
        grid=(N // _MM_B,),
        in_specs=[
            pl.BlockSpec((_MM_B, IN_DIM), lambda i: (i, 0)),
            pl.BlockSpec((IN_DIM, HD), lambda i: (0, 0)),
            pl.BlockSpec((HD, S12), lambda i: (0, 0)),
        ],
        out_specs=[
            pl.BlockSpec((_MM_B, HD), lambda i: (i, 0)),
            pl.BlockSpec((_MM_B, S12), lambda i: (i, 0)),
        ],
        out_shape=[
            jax.ShapeDtypeStruct((N, HD), jnp.float32),
            jax.ShapeDtypeStruct((N, S12), jnp.float32),
        ],
    )(x, W, A12)


def _sc_edge(h_hbm, wex_hbm, src_hbm, dst_hbm, acc_out,
             srcv, dstv, hrow, wrow, zb128,
             acc, sem3):
    c = lax.axis_index("c")
    s = lax.axis_index("s")
    wid = c * NS + s

    zv = jnp.zeros((16,), jnp.float32)

    def _zb(r, carry):
        for j in range(HD // 16):
            zb128[r, pl.ds(j * 16, 16)] = zv
        return carry

    lax.fori_loop(0, 8, _zb, 0)

    base = s * RPT
    nsteps = jnp.where(s == NS - 1, RPT // 8 + 2, RPT // 8)

    def _init(r, carry):
        pltpu.sync_copy(zb128, acc.at[pl.ds(base + r * 8, 8)])
        return carry

    lax.fori_loop(0, nsteps, _init, 0)
    plsc.subcore_barrier()

    ebase = wid * EPW

    def _chunk(k, carry):
        eo = ebase + k * CHUNK
        pltpu.sync_copy(src_hbm.at[pl.ds(eo, CHUNK)], srcv)
        pltpu.sync_copy(dst_hbm.at[pl.ds(eo, CHUNK)], dstv)
        pltpu.sync_copy(wex_hbm.at[pl.ds(eo, CHUNK)], wrow)
        cp3 = pltpu.async_copy(h_hbm.at[srcv], hrow, sem3)
        cp3.wait()

        def _edge(e, icarry):
            for hh in range(H):
                hv = hrow[e, pl.ds(hh * 16, 16)]
                wv = wrow[e, pl.ds(hh * 16, 16)]
                hrow[e, pl.ds(hh * 16, 16)] = hv * wv
            return icarry

        lax.fori_loop(0, CHUNK, _edge, 0)
        pltpu.sync_copy(hrow, acc.at[dstv], add=True)
        return carry

    lax.fori_loop(0, NCHUNK, _chunk, 0)
    plsc.subcore_barrier()

    for j in range(16):
        nrow = 40 if j < 15 else 24
        rs = base + j * 40
        pltpu.sync_copy(acc.at[pl.ds(rs, nrow)], hrow.at[pl.ds(0, nrow)])
        pltpu.sync_copy(hrow.at[pl.ds(0, nrow)], acc_out.at[c, pl.ds(rs, nrow)])

    @pl.when(s == NS - 1)
    def _tail():
        rs = NS * RPT
        pltpu.sync_copy(acc.at[pl.ds(rs, 16)], hrow.at[pl.ds(0, 16)])
        pltpu.sync_copy(hrow.at[pl.ds(0, 16)], acc_out.at[c, pl.ds(rs, 16)])


def _stage2(h, wex, src, dst):
    f = pl.kernel(
        _sc_edge,
        out_type=jax.ShapeDtypeStruct((NC, N, HD), jnp.float32),
        mesh=plsc.VectorSubcoreMesh(
            core_axis_name="c", subcore_axis_name="s",
            num_cores=NC, num_subcores=NS),
        scratch_types=[
            pltpu.VMEM((CHUNK,), jnp.int32),
            pltpu.VMEM((CHUNK,), jnp.int32),
            pltpu.VMEM((CHUNK, HD), jnp.float32),
            pltpu.VMEM((CHUNK, HD), jnp.float32),
            pltpu.VMEM((8, HD), jnp.float32),
            pltpu.VMEM_SHARED((N, HD), jnp.float32),
            pltpu.SemaphoreType.DMA,
        ],
        compiler_params=pltpu.CompilerParams(needs_layout_passes=False),
    )
    return f(h, wex, src, dst)


def _ep_body(acc_ref, den_ref, x_ref, exp_ref, g_ref, b_ref, o_ref):
    a2 = acc_ref[...]
    accb = a2[0] + a2[1]
    denx = jnp.dot(den_ref[...], exp_ref[...],
                   preferred_element_type=jnp.float32)
    denx = jnp.where(denx == 0.0, 1.0, denx)
    y = accb / denx + x_ref[...]
    mu = jnp.mean(y, axis=-1, keepdims=True)
    dd = y - mu
    var = jnp.mean(dd * dd, axis=-1, keepdims=True)
    yn = dd * lax.rsqrt(var + 1e-5) * g_ref[...] + b_ref[...]
    o_ref[...] = jnp.where(yn > 0.0, yn, jnp.exp(jnp.minimum(yn, 0.0)) - 1.0)


_EP_B = 1000


def _stage3(accp, den, x, gamma2, beta2, expand):
    return pl.pallas_call(
        _ep_body,
        grid=(N // _EP_B,),
        in_specs=[
            pl.BlockSpec((NC, _EP_B, HD), lambda i: (0, i, 0)),
            pl.BlockSpec((_EP_B, H), lambda i: (i, 0)),
            pl.BlockSpec((_EP_B, HD), lambda i: (i, 0)),
            pl.BlockSpec((H, HD), lambda i: (0, 0)),
            pl.BlockSpec((1, HD), lambda i: (0, 0)),
            pl.BlockSpec((1, HD), lambda i: (0, 0)),
        ],
        out_specs=pl.BlockSpec((_EP_B, HD), lambda i: (i, 0)),
        out_shape=jax.ShapeDtypeStruct((N, HD), jnp.float32),
    )(accp, den, x, expand, gamma2, beta2)


_EXPAND = np.zeros((H, HD), np.float32)
for _h in range(H):
    _EXPAND[_h, _h * D:(_h + 1) * D] = 1.0


def kernel(x, edge_index, W, a, gamma, beta):
    src = edge_index[0]
    dst = edge_index[1]
    rows = np.arange(HD)
    cols = np.repeat(np.arange(H), D)
    A12 = jnp.zeros((HD, S12), jnp.float32)
    A12 = A12.at[rows, cols].set(jnp.tile(a[:D], H))
    A12 = A12.at[rows, cols + H].set(jnp.tile(a[D:], H))
    h, s12 = _stage1(x, W, A12)
    sg = s12[src, :H] + s12[dst, H:]
    ex = jnp.exp(jnp.where(sg >= 0.0, sg, 0.2 * sg))
    den = jax.ops.segment_sum(ex, dst, num_segments=N)
    wex = jnp.repeat(ex, D, axis=1)
    accp = _stage2(h, wex, src, dst)
    return _stage3(accp, den, x, gamma.reshape(1, HD), beta.reshape(1, HD),
                   jnp.asarray(_EXPAND))
